# Initial kernel scaffold; baseline (speedup 1.0000x reference)
import jax
import jax.numpy as jnp
from jax.experimental import pallas as pl


def kernel(encoded_tokens, pos_table):
    raise NotImplementedError("write your pallas kernel here")


# speedup vs baseline: 1.8005x; 1.8005x over previous
import jax
import jax.numpy as jnp
from jax.experimental import pallas as pl


def _add_kernel(x_ref, p_ref, o_ref):
    o_ref[...] = x_ref[...] + p_ref[...][None, :, :]


def kernel(encoded_tokens, pos_table):
    B, T, D = encoded_tokens.shape
    Tb = 512
    return pl.pallas_call(
        _add_kernel,
        grid=(T // Tb,),
        in_specs=[
            pl.BlockSpec((B, Tb, D), lambda i: (0, i, 0)),
            pl.BlockSpec((Tb, D), lambda i: (i, 0)),
        ],
        out_specs=pl.BlockSpec((B, Tb, D), lambda i: (0, i, 0)),
        out_shape=jax.ShapeDtypeStruct((B, T, D), jnp.float32),
    )(encoded_tokens, pos_table)
